# baseline (speedup 1.0000x reference)
import functools

import jax
import jax.numpy as jnp
from jax import lax
from jax.experimental import pallas as pl
from jax.experimental.pallas import tpu as pltpu
from jax.experimental.pallas import tpu_sc as plsc

F32 = jnp.float32
_DM = 768
_DI = 1536
_DC = 4
_DS = 16
_DTR = 48
_NE = 8
_FFN = 2048
_SEQ = 2048
_EPS = 1e-6
_R = 256


def _silu(v):
    return v * jax.nn.sigmoid(v)


def _k1_body(x_ref, wln_ref, win_ref, xz_ref):
    xr = x_ref[...]
    ms = jnp.mean(xr * xr, axis=1, keepdims=True)
    h = xr * jax.lax.rsqrt(ms + _EPS) * wln_ref[...]
    xz_ref[...] = jnp.dot(h, win_ref[...], preferred_element_type=F32)


def _k2_body(cur_ref, prev_ref, cw_ref, cb_ref, wxp_ref, wdt_ref, bdt_ref,
             xc_ref, dt_ref, dbl_ref):
    pid = pl.program_id(0)
    cur = cur_ref[...]
    prev8 = jnp.where(pid > 0, prev_ref[...], 0.0)
    xfull = jnp.concatenate([prev8, cur], axis=0)
    cw = cw_ref[...]
    xc = cb_ref[...]
    for k in range(_DC):
        s = _DC - 1 - k
        xc = xc + xfull[8 - s:8 - s + _R, :] * cw[k:k + 1, :]
    xc = _silu(xc)
    xc_ref[...] = xc
    dbl = jnp.dot(xc, wxp_ref[...], preferred_element_type=F32)
    dbl_ref[...] = dbl
    dtv = jnp.dot(dbl[:, :_DTR], wdt_ref[...], preferred_element_type=F32)
    dt_ref[...] = jax.nn.softplus(dtv + bdt_ref[...])


_UNROLL = 16


def _k3_body(alogT_ref, d_ref, xc_ref, dt_ref, bc3_ref, y_ref, h_ref):
    nd = xc_ref.shape[1]
    tb = pl.program_id(0)
    nt = xc_ref.shape[0]
    a = -jnp.exp(alogT_ref[...])
    dcoef = d_ref[...]

    @pl.when(tb == 0)
    def _():
        h_ref[...] = jnp.zeros((_DS, nd), F32)

    def blk(i, h):
        t0 = i * _UNROLL
        dts = dt_ref[pl.ds(t0, _UNROLL), :]
        xs = xc_ref[pl.ds(t0, _UNROLL), :]
        dtx = dts * xs
        bcs = bc3_ref[pl.ds(t0, _UNROLL)]
        ys = []
        for j in range(_UNROLL):
            dA = jnp.exp(dts[j:j + 1, :] * a)
            h = dA * h + jnp.dot(bcs[j, :_DS], dtx[j:j + 1, :], preferred_element_type=F32)
            ys.append(jnp.sum(h * bcs[j, _DS:], axis=0, keepdims=True))
        y_ref[pl.ds(t0, _UNROLL), :] = jnp.concatenate(ys, axis=0) + xs * dcoef
        return h

    h_ref[...] = jax.lax.fori_loop(0, nt // _UNROLL, blk, h_ref[...])


def _k4_body(y_ref, res_ref, x_ref, wln2_ref, wout_ref, wr_ref,
             out1_ref, h2b_ref, logits_ref, ri_ref):
    g = y_ref[...] * _silu(res_ref[...])
    xo = jnp.dot(g, wout_ref[...], preferred_element_type=F32)
    x1 = x_ref[...] + xo
    out1_ref[...] = x1
    ms = jnp.mean(x1 * x1, axis=1, keepdims=True)
    h2 = x1 * jax.lax.rsqrt(ms + _EPS) * wln2_ref[...]
    h2b_ref[...] = h2
    lg = jnp.dot(h2, wr_ref[...], preferred_element_type=F32)
    logits_ref[...] = lg
    lane = jax.lax.broadcasted_iota(jnp.int32, lg.shape, 1)
    neg = jnp.where(lane < _NE, lg, -1e30)
    m = jnp.max(neg, axis=1, keepdims=True)
    ex = jnp.exp(neg - m)
    p = ex / jnp.sum(ex, axis=1, keepdims=True)
    v1 = jnp.max(p, axis=1, keepdims=True)
    i1 = jnp.min(jnp.where(p >= v1, lane, 999), axis=1, keepdims=True)
    p2 = jnp.where(lane == i1, -1.0, p)
    v2 = jnp.max(p2, axis=1, keepdims=True)
    i2 = jnp.min(jnp.where(p2 >= v2, lane, 999), axis=1, keepdims=True)
    ri_ref[...] = (jnp.where(lane == 0, i1.astype(F32), 0.0)
                   + jnp.where(lane == 1, i2.astype(F32), 0.0)
                   + jnp.where(lane == 2, v1, 0.0)
                   + jnp.where(lane == 3, v2, 0.0))


_NP = 6144
_NT = _NP // _R


def _sel(mat, idx, lane):
    return jnp.sum(jnp.where(lane == idx, mat, 0.0), axis=1, keepdims=True)


def _k4b_body(ri_ref, posv_ref, te_ref, p1s_ref, p2s_ref):
    ri = ri_ref[...]
    lane = jax.lax.broadcasted_iota(jnp.int32, (_R, 128), 1).astype(F32)
    lseq = jax.lax.broadcasted_iota(jnp.int32, (_SEQ, 128), 1).astype(F32)
    i1 = ri[:, 0:1]
    i2 = ri[:, 1:2]
    o1 = jnp.where((lseq == i1) & (lseq < _NE), 1.0, 0.0)
    o2 = jnp.where((lseq == i2) & (lseq < _NE), 1.0, 0.0)
    ri_blk = jax.lax.broadcasted_iota(jnp.int32, (_R, _R), 0)
    ci_blk = jax.lax.broadcasted_iota(jnp.int32, (_R, _R), 1)
    ltri = jnp.where(ri_blk > ci_blk, 1.0, 0.0)
    c1 = jnp.zeros((1, 128), F32)
    c2 = jnp.zeros((1, 128), F32)
    nblk = _SEQ // _R
    for b in range(nblk):
        rows = slice(b * _R, (b + 1) * _R)
        o1b = o1[rows]
        o2b = o2[rows]
        p1s_ref[rows, :] = jnp.dot(ltri, o1b, preferred_element_type=F32) + c1
        p2s_ref[rows, :] = jnp.dot(ltri, o2b, preferred_element_type=F32) + c2
        c1 = c1 + jnp.sum(o1b, axis=0, keepdims=True)
        c2 = c2 + jnp.sum(o2b, axis=0, keepdims=True)
    cnt = c1 + c2
    rc = jnp.ceil(cnt / _R) * _R
    e_row = jax.lax.broadcasted_iota(jnp.int32, (128, 128), 0)
    e_col = jax.lax.broadcasted_iota(jnp.int32, (128, 128), 1)
    ustri = jnp.where(e_row < e_col, 1.0, 0.0)
    start = jnp.dot(rc, ustri, preferred_element_type=F32)
    for b in range(nblk):
        rows = slice(b * _R, (b + 1) * _R)
        i1b = i1[rows]
        i2b = i2[rows]
        p1b = p1s_ref[rows, :]
        p2b = p2s_ref[rows, :]
        pos1 = _sel(start, i1b, lane) + _sel(p1b, i1b, lane) + _sel(p2b, i1b, lane)
        pos2 = _sel(start, i2b, lane) + _sel(p1b, i2b, lane) + _sel(p2b, i2b, lane)
        posv_ref[rows, :] = (jnp.where(lane == 0, pos1, 0.0)
                             + jnp.where(lane == 1, pos2, 0.0)).astype(jnp.int32)
    jrow = jax.lax.broadcasted_iota(jnp.int32, (32, 128), 0).astype(F32) * float(_R)
    lane32 = jax.lax.broadcasted_iota(jnp.int32, (32, 128), 1).astype(F32)
    cond = ((jrow >= start) & (jrow < start + rc) & (lane32 < _NE)).astype(F32)
    teval = jnp.sum(lane32 * cond, axis=1, keepdims=True)
    anyc = jnp.sum(cond, axis=1, keepdims=True)
    teval = jnp.where(anyc > 0, teval, float(_NE - 1))
    te_ref[...] = jnp.where(lane32 == 0, teval, 0.0).astype(jnp.int32)


_NWORK = 32
_CHUNK = _SEQ // _NWORK


def _sc_dispatch(h2b, posi):
    mesh = plsc.VectorSubcoreMesh(core_axis_name="c", subcore_axis_name="s")

    @functools.partial(
        pl.kernel,
        out_type=jax.ShapeDtypeStruct((_NP, _DM), F32),
        mesh=mesh,
        scratch_types=[
            pltpu.VMEM((_CHUNK,), jnp.int32),
            pltpu.VMEM((_CHUNK, _DM), F32),
        ],
    )
    def k(h2_hbm, pos_hbm, xs_hbm, idx_v, rows_v):
        wid = lax.axis_index("s") * 2 + lax.axis_index("c")
        base = wid * _CHUNK
        pltpu.sync_copy(h2_hbm.at[pl.ds(base, _CHUNK)], rows_v)
        pltpu.sync_copy(pos_hbm.at[wid, 0], idx_v)
        pltpu.sync_copy(rows_v, xs_hbm.at[idx_v])
        pltpu.sync_copy(pos_hbm.at[wid, 1], idx_v)
        pltpu.sync_copy(rows_v, xs_hbm.at[idx_v])

    return k(h2b, posi)


_GCHUNK = 2 * _SEQ // _NWORK


def _sc_combine_gather(rows, posg):
    mesh = plsc.VectorSubcoreMesh(core_axis_name="c", subcore_axis_name="s")

    @functools.partial(
        pl.kernel,
        out_type=jax.ShapeDtypeStruct((2 * _SEQ, _DM), F32),
        mesh=mesh,
        scratch_types=[
            pltpu.VMEM((_GCHUNK,), jnp.int32),
            pltpu.VMEM((_GCHUNK, _DM), F32),
            pltpu.SemaphoreType.DMA,
        ],
    )
    def k(rows_hbm, idx_hbm, out_hbm, idx_v, rows_v, sem):
        wid = lax.axis_index("s") * 2 + lax.axis_index("c")
        base = wid * _GCHUNK
        pltpu.sync_copy(idx_hbm.at[pl.ds(base, _GCHUNK)], idx_v)
        pltpu.async_copy(rows_hbm.at[idx_v], rows_v, sem).wait()
        pltpu.sync_copy(rows_v, out_hbm.at[pl.ds(base, _GCHUNK)])

    return k(rows, posg)


def _k5_body(te_ref, xs_ref, wg_ref, wu_ref, wd_ref, rows_ref):
    bf = jnp.bfloat16
    xs = xs_ref[...].astype(bf)
    hg = _silu(jnp.dot(xs, wg_ref[0].astype(bf), preferred_element_type=F32))
    hu = jnp.dot(xs, wu_ref[0].astype(bf), preferred_element_type=F32)
    pe = jnp.dot((hg * hu).astype(bf), wd_ref[0].astype(bf),
                 preferred_element_type=F32)
    rows_ref[...] = pe


def _k6_body(c1_ref, c2_ref, out1_ref, ri_ref, out_ref):
    ri = ri_ref[...]
    lane = jax.lax.broadcasted_iota(jnp.int32, ri.shape, 1)
    g1 = jnp.sum(jnp.where(lane == 2, ri, 0.0), axis=1, keepdims=True)
    g2 = jnp.sum(jnp.where(lane == 3, ri, 0.0), axis=1, keepdims=True)
    out_ref[...] = out1_ref[...] + g1 * c1_ref[...] + g2 * c2_ref[...]


def kernel(x, w_ln1, w_ln2, W_in, conv_w, conv_b, W_xproj, W_dt, b_dt,
           A_log, D, W_out, W_r, Wg, Wu, Wd):
    xf = x.reshape(_SEQ, _DM)
    wln1 = w_ln1.reshape(1, _DM)
    wln2 = w_ln2.reshape(1, _DM)
    nt = _SEQ // _R

    xz = pl.pallas_call(
        _k1_body,
        grid=(nt,),
        in_specs=[
            pl.BlockSpec((_R, _DM), lambda i: (i, 0)),
            pl.BlockSpec((1, _DM), lambda i: (0, 0)),
            pl.BlockSpec((_DM, 2 * _DI), lambda i: (0, 0)),
        ],
        out_specs=pl.BlockSpec((_R, 2 * _DI), lambda i: (i, 0)),
        out_shape=jax.ShapeDtypeStruct((_SEQ, 2 * _DI), F32),
    )(xf, wln1, W_in)

    cwT = conv_w.T
    cb = conv_b.reshape(1, _DI)
    wxp = jnp.pad(W_xproj, ((0, 0), (0, 128 - (_DTR + 2 * _DS))))
    bdt = b_dt.reshape(1, _DI)
    xc, dt, dbl = pl.pallas_call(
        _k2_body,
        grid=(nt,),
        in_specs=[
            pl.BlockSpec((_R, _DI), lambda i: (i, 0)),
            pl.BlockSpec((8, _DI), lambda i: (jnp.maximum(i * (_R // 8) - 1, 0), 0)),
            pl.BlockSpec((_DC, _DI), lambda i: (0, 0)),
            pl.BlockSpec((1, _DI), lambda i: (0, 0)),
            pl.BlockSpec((_DI, 128), lambda i: (0, 0)),
            pl.BlockSpec((_DTR, _DI), lambda i: (0, 0)),
            pl.BlockSpec((1, _DI), lambda i: (0, 0)),
        ],
        out_specs=[
            pl.BlockSpec((_R, _DI), lambda i: (i, 0)),
            pl.BlockSpec((_R, _DI), lambda i: (i, 0)),
            pl.BlockSpec((_R, 128), lambda i: (i, 0)),
        ],
        out_shape=[
            jax.ShapeDtypeStruct((_SEQ, _DI), F32),
            jax.ShapeDtypeStruct((_SEQ, _DI), F32),
            jax.ShapeDtypeStruct((_SEQ, 128), F32),
        ],
    )(xz, xz, cwT, cb, wxp, W_dt, bdt)

    bc3 = dbl[:, _DTR:_DTR + 2 * _DS].reshape(_SEQ, 2 * _DS, 1)
    alogT = A_log.T
    dvec = D.reshape(1, _DI)
    y = pl.pallas_call(
        _k3_body,
        grid=(_SEQ // _R,),
        in_specs=[
            pl.BlockSpec((_DS, _DI), lambda j: (0, 0)),
            pl.BlockSpec((1, _DI), lambda j: (0, 0)),
            pl.BlockSpec((_R, _DI), lambda j: (j, 0)),
            pl.BlockSpec((_R, _DI), lambda j: (j, 0)),
            pl.BlockSpec((_R, 2 * _DS, 1), lambda j: (j, 0, 0)),
        ],
        out_specs=pl.BlockSpec((_R, _DI), lambda j: (j, 0)),
        out_shape=jax.ShapeDtypeStruct((_SEQ, _DI), F32),
        scratch_shapes=[pltpu.VMEM((_DS, _DI), F32)],
    )(alogT, dvec, xc, dt, bc3)

    wrp = jnp.pad(W_r, ((0, 0), (0, 128 - _NE)))
    out1, h2, logits, ri = pl.pallas_call(
        _k4_body,
        grid=(nt,),
        in_specs=[
            pl.BlockSpec((_R, _DI), lambda i: (i, 0)),
            pl.BlockSpec((_R, _DI), lambda i: (i, 1)),
            pl.BlockSpec((_R, _DM), lambda i: (i, 0)),
            pl.BlockSpec((1, _DM), lambda i: (0, 0)),
            pl.BlockSpec((_DI, _DM), lambda i: (0, 0)),
            pl.BlockSpec((_DM, 128), lambda i: (0, 0)),
        ],
        out_specs=[
            pl.BlockSpec((_R, _DM), lambda i: (i, 0)),
            pl.BlockSpec((_R, _DM), lambda i: (i, 0)),
            pl.BlockSpec((_R, 128), lambda i: (i, 0)),
            pl.BlockSpec((_R, 128), lambda i: (i, 0)),
        ],
        out_shape=[
            jax.ShapeDtypeStruct((_SEQ, _DM), F32),
            jax.ShapeDtypeStruct((_SEQ, _DM), F32),
            jax.ShapeDtypeStruct((_SEQ, 128), F32),
            jax.ShapeDtypeStruct((_SEQ, 128), F32),
        ],
    )(y, xz, xf, wln2, W_out, wrp)

    posv, teo = pl.pallas_call(
        _k4b_body,
        grid=(1,),
        in_specs=[pl.BlockSpec((_SEQ, 128), lambda i: (0, 0))],
        out_specs=[
            pl.BlockSpec((_SEQ, 128), lambda i: (0, 0)),
            pl.BlockSpec((32, 128), lambda i: (0, 0)),
        ],
        out_shape=[
            jax.ShapeDtypeStruct((_SEQ, 128), jnp.int32),
            jax.ShapeDtypeStruct((32, 128), jnp.int32),
        ],
        scratch_shapes=[pltpu.VMEM((_SEQ, 128), F32),
                        pltpu.VMEM((_SEQ, 128), F32)],
    )(ri)

    pos1i = posv[:, 0]
    pos2i = posv[:, 1]
    posi = jnp.stack([pos1i.reshape(_NWORK, _CHUNK),
                      pos2i.reshape(_NWORK, _CHUNK)], axis=1)
    posg = jnp.concatenate([pos1i, pos2i])
    te_arr = teo[:_NT, 0]

    xs = _sc_dispatch(h2, posi)

    rows = pl.pallas_call(
        _k5_body,
        grid_spec=pltpu.PrefetchScalarGridSpec(
            num_scalar_prefetch=1,
            grid=(_NT,),
            in_specs=[
                pl.BlockSpec((_R, _DM), lambda i, te: (i, 0)),
                pl.BlockSpec((1, _DM, _FFN), lambda i, te: (te[i], 0, 0)),
                pl.BlockSpec((1, _DM, _FFN), lambda i, te: (te[i], 0, 0)),
                pl.BlockSpec((1, _FFN, _DM), lambda i, te: (te[i], 0, 0)),
            ],
            out_specs=pl.BlockSpec((_R, _DM), lambda i, te: (i, 0)),
        ),
        out_shape=jax.ShapeDtypeStruct((_NP, _DM), F32),
    )(te_arr, xs, Wg, Wu, Wd)

    comb = _sc_combine_gather(rows, posg)

    out = pl.pallas_call(
        _k6_body,
        grid=(nt,),
        in_specs=[
            pl.BlockSpec((_R, _DM), lambda i: (i, 0)),
            pl.BlockSpec((_R, _DM), lambda i: (i + _SEQ // _R, 0)),
            pl.BlockSpec((_R, _DM), lambda i: (i, 0)),
            pl.BlockSpec((_R, 128), lambda i: (i, 0)),
        ],
        out_specs=pl.BlockSpec((_R, _DM), lambda i: (i, 0)),
        out_shape=jax.ShapeDtypeStruct((_SEQ, _DM), F32),
    )(comb, comb, out1, ri)

    return out.reshape(1, _SEQ, _DM), logits[:, :_NE]


# speedup vs baseline: 1.0051x; 1.0051x over previous
import functools

import jax
import jax.numpy as jnp
from jax import lax
from jax.experimental import pallas as pl
from jax.experimental.pallas import tpu as pltpu
from jax.experimental.pallas import tpu_sc as plsc

F32 = jnp.float32
_DM = 768
_DI = 1536
_DC = 4
_DS = 16
_DTR = 48
_NE = 8
_FFN = 2048
_SEQ = 2048
_EPS = 1e-6
_R = 256


def _silu(v):
    return v * jax.nn.sigmoid(v)


def _k1_body(x_ref, wln_ref, win_ref, xz_ref):
    xr = x_ref[...]
    ms = jnp.mean(xr * xr, axis=1, keepdims=True)
    h = xr * jax.lax.rsqrt(ms + _EPS) * wln_ref[...]
    xz_ref[...] = jnp.dot(h, win_ref[...], preferred_element_type=F32)


def _k2_body(cur_ref, prev_ref, cw_ref, cb_ref, wxp_ref, wdt_ref, bdt_ref,
             xc_ref, dt_ref, dbl_ref):
    pid = pl.program_id(0)
    cur = cur_ref[...]
    prev8 = jnp.where(pid > 0, prev_ref[...], 0.0)
    xfull = jnp.concatenate([prev8, cur], axis=0)
    cw = cw_ref[...]
    xc = cb_ref[...]
    for k in range(_DC):
        s = _DC - 1 - k
        xc = xc + xfull[8 - s:8 - s + _R, :] * cw[k:k + 1, :]
    xc = _silu(xc)
    xc_ref[...] = xc
    dbl = jnp.dot(xc, wxp_ref[...], preferred_element_type=F32)
    dbl_ref[...] = dbl
    dtv = jnp.dot(dbl[:, :_DTR], wdt_ref[...], preferred_element_type=F32)
    dt_ref[...] = jax.nn.softplus(dtv + bdt_ref[...])


_UNROLL = 16


def _k3_body(alogT_ref, d_ref, xc_ref, dt_ref, bc3_ref, y_ref, h_ref):
    nd = xc_ref.shape[1]
    tb = pl.program_id(0)
    nt = xc_ref.shape[0]
    a = -jnp.exp(alogT_ref[...])
    dcoef = d_ref[...]

    @pl.when(tb == 0)
    def _():
        h_ref[...] = jnp.zeros((_DS, nd), F32)

    def blk(i, h):
        t0 = i * _UNROLL
        dts = dt_ref[pl.ds(t0, _UNROLL), :]
        xs = xc_ref[pl.ds(t0, _UNROLL), :]
        dtx = dts * xs
        bcs = bc3_ref[pl.ds(t0, _UNROLL)]
        ys = []
        for j in range(_UNROLL):
            dA = jnp.exp(dts[j:j + 1, :] * a)
            h = dA * h + dtx[j:j + 1, :] * bcs[j, :_DS]
            ys.append(jnp.sum(h * bcs[j, _DS:], axis=0, keepdims=True))
        y_ref[pl.ds(t0, _UNROLL), :] = jnp.concatenate(ys, axis=0) + xs * dcoef
        return h

    h_ref[...] = jax.lax.fori_loop(0, nt // _UNROLL, blk, h_ref[...])


def _k4_body(y_ref, res_ref, x_ref, wln2_ref, wout_ref, wr_ref,
             out1_ref, h2b_ref, logits_ref, ri_ref):
    g = y_ref[...] * _silu(res_ref[...])
    xo = jnp.dot(g, wout_ref[...], preferred_element_type=F32)
    x1 = x_ref[...] + xo
    out1_ref[...] = x1
    ms = jnp.mean(x1 * x1, axis=1, keepdims=True)
    h2 = x1 * jax.lax.rsqrt(ms + _EPS) * wln2_ref[...]
    h2b_ref[...] = h2
    lg = jnp.dot(h2, wr_ref[...], preferred_element_type=F32)
    logits_ref[...] = lg
    lane = jax.lax.broadcasted_iota(jnp.int32, lg.shape, 1)
    neg = jnp.where(lane < _NE, lg, -1e30)
    m = jnp.max(neg, axis=1, keepdims=True)
    ex = jnp.exp(neg - m)
    p = ex / jnp.sum(ex, axis=1, keepdims=True)
    v1 = jnp.max(p, axis=1, keepdims=True)
    i1 = jnp.min(jnp.where(p >= v1, lane, 999), axis=1, keepdims=True)
    p2 = jnp.where(lane == i1, -1.0, p)
    v2 = jnp.max(p2, axis=1, keepdims=True)
    i2 = jnp.min(jnp.where(p2 >= v2, lane, 999), axis=1, keepdims=True)
    ri_ref[...] = (jnp.where(lane == 0, i1.astype(F32), 0.0)
                   + jnp.where(lane == 1, i2.astype(F32), 0.0)
                   + jnp.where(lane == 2, v1, 0.0)
                   + jnp.where(lane == 3, v2, 0.0))


_NP = 6144
_NT = _NP // _R


def _sel(mat, idx, lane):
    return jnp.sum(jnp.where(lane == idx, mat, 0.0), axis=1, keepdims=True)


def _k4b_body(ri_ref, posv_ref, te_ref, p1s_ref, p2s_ref):
    ri = ri_ref[...]
    lane = jax.lax.broadcasted_iota(jnp.int32, (_R, 128), 1).astype(F32)
    lseq = jax.lax.broadcasted_iota(jnp.int32, (_SEQ, 128), 1).astype(F32)
    i1 = ri[:, 0:1]
    i2 = ri[:, 1:2]
    o1 = jnp.where((lseq == i1) & (lseq < _NE), 1.0, 0.0)
    o2 = jnp.where((lseq == i2) & (lseq < _NE), 1.0, 0.0)
    ri_blk = jax.lax.broadcasted_iota(jnp.int32, (_R, _R), 0)
    ci_blk = jax.lax.broadcasted_iota(jnp.int32, (_R, _R), 1)
    ltri = jnp.where(ri_blk > ci_blk, 1.0, 0.0)
    c1 = jnp.zeros((1, 128), F32)
    c2 = jnp.zeros((1, 128), F32)
    nblk = _SEQ // _R
    for b in range(nblk):
        rows = slice(b * _R, (b + 1) * _R)
        o1b = o1[rows]
        o2b = o2[rows]
        p1s_ref[rows, :] = jnp.dot(ltri, o1b, preferred_element_type=F32) + c1
        p2s_ref[rows, :] = jnp.dot(ltri, o2b, preferred_element_type=F32) + c2
        c1 = c1 + jnp.sum(o1b, axis=0, keepdims=True)
        c2 = c2 + jnp.sum(o2b, axis=0, keepdims=True)
    cnt = c1 + c2
    rc = jnp.ceil(cnt / _R) * _R
    e_row = jax.lax.broadcasted_iota(jnp.int32, (128, 128), 0)
    e_col = jax.lax.broadcasted_iota(jnp.int32, (128, 128), 1)
    ustri = jnp.where(e_row < e_col, 1.0, 0.0)
    start = jnp.dot(rc, ustri, preferred_element_type=F32)
    for b in range(nblk):
        rows = slice(b * _R, (b + 1) * _R)
        i1b = i1[rows]
        i2b = i2[rows]
        p1b = p1s_ref[rows, :]
        p2b = p2s_ref[rows, :]
        pos1 = _sel(start, i1b, lane) + _sel(p1b, i1b, lane) + _sel(p2b, i1b, lane)
        pos2 = _sel(start, i2b, lane) + _sel(p1b, i2b, lane) + _sel(p2b, i2b, lane)
        posv_ref[rows, :] = (jnp.where(lane == 0, pos1, 0.0)
                             + jnp.where(lane == 1, pos2, 0.0)).astype(jnp.int32)
    jrow = jax.lax.broadcasted_iota(jnp.int32, (32, 128), 0).astype(F32) * float(_R)
    lane32 = jax.lax.broadcasted_iota(jnp.int32, (32, 128), 1).astype(F32)
    cond = ((jrow >= start) & (jrow < start + rc) & (lane32 < _NE)).astype(F32)
    teval = jnp.sum(lane32 * cond, axis=1, keepdims=True)
    anyc = jnp.sum(cond, axis=1, keepdims=True)
    teval = jnp.where(anyc > 0, teval, float(_NE - 1))
    te_ref[...] = jnp.where(lane32 == 0, teval, 0.0).astype(jnp.int32)


_NWORK = 32
_CHUNK = _SEQ // _NWORK


def _sc_dispatch(h2b, posi):
    mesh = plsc.VectorSubcoreMesh(core_axis_name="c", subcore_axis_name="s")

    @functools.partial(
        pl.kernel,
        out_type=jax.ShapeDtypeStruct((_NP, _DM), F32),
        mesh=mesh,
        scratch_types=[
            pltpu.VMEM((_CHUNK,), jnp.int32),
            pltpu.VMEM((_CHUNK, _DM), F32),
        ],
    )
    def k(h2_hbm, pos_hbm, xs_hbm, idx_v, rows_v):
        wid = lax.axis_index("s") * 2 + lax.axis_index("c")
        base = wid * _CHUNK
        pltpu.sync_copy(h2_hbm.at[pl.ds(base, _CHUNK)], rows_v)
        pltpu.sync_copy(pos_hbm.at[wid, 0], idx_v)
        pltpu.sync_copy(rows_v, xs_hbm.at[idx_v])
        pltpu.sync_copy(pos_hbm.at[wid, 1], idx_v)
        pltpu.sync_copy(rows_v, xs_hbm.at[idx_v])

    return k(h2b, posi)


_GCHUNK = 2 * _SEQ // _NWORK


def _sc_combine_gather(rows, posg):
    mesh = plsc.VectorSubcoreMesh(core_axis_name="c", subcore_axis_name="s")

    @functools.partial(
        pl.kernel,
        out_type=jax.ShapeDtypeStruct((2 * _SEQ, _DM), F32),
        mesh=mesh,
        scratch_types=[
            pltpu.VMEM((_GCHUNK,), jnp.int32),
            pltpu.VMEM((_GCHUNK, _DM), F32),
            pltpu.SemaphoreType.DMA,
        ],
    )
    def k(rows_hbm, idx_hbm, out_hbm, idx_v, rows_v, sem):
        wid = lax.axis_index("s") * 2 + lax.axis_index("c")
        base = wid * _GCHUNK
        pltpu.sync_copy(idx_hbm.at[pl.ds(base, _GCHUNK)], idx_v)
        pltpu.async_copy(rows_hbm.at[idx_v], rows_v, sem).wait()
        pltpu.sync_copy(rows_v, out_hbm.at[pl.ds(base, _GCHUNK)])

    return k(rows, posg)


def _k5_body(te_ref, xs_ref, wg_ref, wu_ref, wd_ref, rows_ref):
    bf = jnp.bfloat16
    xs = xs_ref[...].astype(bf)
    hg = _silu(jnp.dot(xs, wg_ref[0].astype(bf), preferred_element_type=F32))
    hu = jnp.dot(xs, wu_ref[0].astype(bf), preferred_element_type=F32)
    pe = jnp.dot((hg * hu).astype(bf), wd_ref[0].astype(bf),
                 preferred_element_type=F32)
    rows_ref[...] = pe


def _k6_body(c1_ref, c2_ref, out1_ref, ri_ref, out_ref):
    ri = ri_ref[...]
    lane = jax.lax.broadcasted_iota(jnp.int32, ri.shape, 1)
    g1 = jnp.sum(jnp.where(lane == 2, ri, 0.0), axis=1, keepdims=True)
    g2 = jnp.sum(jnp.where(lane == 3, ri, 0.0), axis=1, keepdims=True)
    out_ref[...] = out1_ref[...] + g1 * c1_ref[...] + g2 * c2_ref[...]


def kernel(x, w_ln1, w_ln2, W_in, conv_w, conv_b, W_xproj, W_dt, b_dt,
           A_log, D, W_out, W_r, Wg, Wu, Wd):
    xf = x.reshape(_SEQ, _DM)
    wln1 = w_ln1.reshape(1, _DM)
    wln2 = w_ln2.reshape(1, _DM)
    nt = _SEQ // _R

    xz = pl.pallas_call(
        _k1_body,
        grid=(nt,),
        in_specs=[
            pl.BlockSpec((_R, _DM), lambda i: (i, 0)),
            pl.BlockSpec((1, _DM), lambda i: (0, 0)),
            pl.BlockSpec((_DM, 2 * _DI), lambda i: (0, 0)),
        ],
        out_specs=pl.BlockSpec((_R, 2 * _DI), lambda i: (i, 0)),
        out_shape=jax.ShapeDtypeStruct((_SEQ, 2 * _DI), F32),
    )(xf, wln1, W_in)

    cwT = conv_w.T
    cb = conv_b.reshape(1, _DI)
    wxp = jnp.pad(W_xproj, ((0, 0), (0, 128 - (_DTR + 2 * _DS))))
    bdt = b_dt.reshape(1, _DI)
    xc, dt, dbl = pl.pallas_call(
        _k2_body,
        grid=(nt,),
        in_specs=[
            pl.BlockSpec((_R, _DI), lambda i: (i, 0)),
            pl.BlockSpec((8, _DI), lambda i: (jnp.maximum(i * (_R // 8) - 1, 0), 0)),
            pl.BlockSpec((_DC, _DI), lambda i: (0, 0)),
            pl.BlockSpec((1, _DI), lambda i: (0, 0)),
            pl.BlockSpec((_DI, 128), lambda i: (0, 0)),
            pl.BlockSpec((_DTR, _DI), lambda i: (0, 0)),
            pl.BlockSpec((1, _DI), lambda i: (0, 0)),
        ],
        out_specs=[
            pl.BlockSpec((_R, _DI), lambda i: (i, 0)),
            pl.BlockSpec((_R, _DI), lambda i: (i, 0)),
            pl.BlockSpec((_R, 128), lambda i: (i, 0)),
        ],
        out_shape=[
            jax.ShapeDtypeStruct((_SEQ, _DI), F32),
            jax.ShapeDtypeStruct((_SEQ, _DI), F32),
            jax.ShapeDtypeStruct((_SEQ, 128), F32),
        ],
    )(xz, xz, cwT, cb, wxp, W_dt, bdt)

    bc3 = dbl[:, _DTR:_DTR + 2 * _DS].reshape(_SEQ, 2 * _DS, 1)
    alogT = A_log.T
    dvec = D.reshape(1, _DI)
    y = pl.pallas_call(
        _k3_body,
        grid=(_SEQ // _R,),
        in_specs=[
            pl.BlockSpec((_DS, _DI), lambda j: (0, 0)),
            pl.BlockSpec((1, _DI), lambda j: (0, 0)),
            pl.BlockSpec((_R, _DI), lambda j: (j, 0)),
            pl.BlockSpec((_R, _DI), lambda j: (j, 0)),
            pl.BlockSpec((_R, 2 * _DS, 1), lambda j: (j, 0, 0)),
        ],
        out_specs=pl.BlockSpec((_R, _DI), lambda j: (j, 0)),
        out_shape=jax.ShapeDtypeStruct((_SEQ, _DI), F32),
        scratch_shapes=[pltpu.VMEM((_DS, _DI), F32)],
    )(alogT, dvec, xc, dt, bc3)

    wrp = jnp.pad(W_r, ((0, 0), (0, 128 - _NE)))
    out1, h2, logits, ri = pl.pallas_call(
        _k4_body,
        grid=(nt,),
        in_specs=[
            pl.BlockSpec((_R, _DI), lambda i: (i, 0)),
            pl.BlockSpec((_R, _DI), lambda i: (i, 1)),
            pl.BlockSpec((_R, _DM), lambda i: (i, 0)),
            pl.BlockSpec((1, _DM), lambda i: (0, 0)),
            pl.BlockSpec((_DI, _DM), lambda i: (0, 0)),
            pl.BlockSpec((_DM, 128), lambda i: (0, 0)),
        ],
        out_specs=[
            pl.BlockSpec((_R, _DM), lambda i: (i, 0)),
            pl.BlockSpec((_R, _DM), lambda i: (i, 0)),
            pl.BlockSpec((_R, 128), lambda i: (i, 0)),
            pl.BlockSpec((_R, 128), lambda i: (i, 0)),
        ],
        out_shape=[
            jax.ShapeDtypeStruct((_SEQ, _DM), F32),
            jax.ShapeDtypeStruct((_SEQ, _DM), F32),
            jax.ShapeDtypeStruct((_SEQ, 128), F32),
            jax.ShapeDtypeStruct((_SEQ, 128), F32),
        ],
    )(y, xz, xf, wln2, W_out, wrp)

    posv, teo = pl.pallas_call(
        _k4b_body,
        grid=(1,),
        in_specs=[pl.BlockSpec((_SEQ, 128), lambda i: (0, 0))],
        out_specs=[
            pl.BlockSpec((_SEQ, 128), lambda i: (0, 0)),
            pl.BlockSpec((32, 128), lambda i: (0, 0)),
        ],
        out_shape=[
            jax.ShapeDtypeStruct((_SEQ, 128), jnp.int32),
            jax.ShapeDtypeStruct((32, 128), jnp.int32),
        ],
        scratch_shapes=[pltpu.VMEM((_SEQ, 128), F32),
                        pltpu.VMEM((_SEQ, 128), F32)],
    )(ri)

    pos1i = posv[:, 0]
    pos2i = posv[:, 1]
    posi = jnp.stack([pos1i.reshape(_NWORK, _CHUNK),
                      pos2i.reshape(_NWORK, _CHUNK)], axis=1)
    posg = jnp.concatenate([pos1i, pos2i])
    te_arr = teo[:_NT, 0]

    xs = _sc_dispatch(h2, posi)

    rows = pl.pallas_call(
        _k5_body,
        grid_spec=pltpu.PrefetchScalarGridSpec(
            num_scalar_prefetch=1,
            grid=(_NT,),
            in_specs=[
                pl.BlockSpec((_R, _DM), lambda i, te: (i, 0)),
                pl.BlockSpec((1, _DM, _FFN), lambda i, te: (te[i], 0, 0)),
                pl.BlockSpec((1, _DM, _FFN), lambda i, te: (te[i], 0, 0)),
                pl.BlockSpec((1, _FFN, _DM), lambda i, te: (te[i], 0, 0)),
            ],
            out_specs=pl.BlockSpec((_R, _DM), lambda i, te: (i, 0)),
        ),
        out_shape=jax.ShapeDtypeStruct((_NP, _DM), F32),
    )(te_arr, xs, Wg, Wu, Wd)

    comb = _sc_combine_gather(rows, posg)

    out = pl.pallas_call(
        _k6_body,
        grid=(nt,),
        in_specs=[
            pl.BlockSpec((_R, _DM), lambda i: (i, 0)),
            pl.BlockSpec((_R, _DM), lambda i: (i + _SEQ // _R, 0)),
            pl.BlockSpec((_R, _DM), lambda i: (i, 0)),
            pl.BlockSpec((_R, 128), lambda i: (i, 0)),
        ],
        out_specs=pl.BlockSpec((_R, _DM), lambda i: (i, 0)),
        out_shape=jax.ShapeDtypeStruct((_SEQ, _DM), F32),
    )(comb, comb, out1, ri)

    return out.reshape(1, _SEQ, _DM), logits[:, :_NE]
